# baseline (speedup 1.0000x reference)
import jax
import jax.numpy as jnp
from jax.experimental import pallas as pl
from jax.experimental.pallas import tpu as pltpu

BN = 1024


def _mm_kernel(scale_ref, x_ref, w_ref, o_ref):
    acc = jax.lax.dot_general(
        x_ref[...],
        w_ref[...],
        (((1,), (1,)), ((), ())),
        preferred_element_type=jnp.float32,
    )
    o_ref[...] = acc * scale_ref[0]


@jax.jit
def kernel(x, W, scale):
    M, K = x.shape
    N = W.shape[0]
    return pl.pallas_call(
        _mm_kernel,
        grid_spec=pltpu.PrefetchScalarGridSpec(
            num_scalar_prefetch=1,
            grid=(N // BN,),
            in_specs=[
                pl.BlockSpec((M, K), lambda n, *_: (0, 0)),
                pl.BlockSpec((BN, K), lambda n, *_: (n, 0)),
            ],
            out_specs=pl.BlockSpec((M, BN), lambda n, *_: (0, n)),
        ),
        out_shape=jax.ShapeDtypeStruct((M, N), jnp.float32),
        compiler_params=pltpu.CompilerParams(
            dimension_semantics=("parallel",),
            vmem_limit_bytes=100 * 1024 * 1024,
        ),
    )(scale, x, W)


# speedup vs baseline: 1.0165x; 1.0165x over previous
import jax
import jax.numpy as jnp
from jax.experimental import pallas as pl
from jax.experimental.pallas import tpu as pltpu

BN = 1024
NCHUNK = 4


def _mm_kernel(scale_ref, x_hbm, w_ref, o_ref, x_vmem, sems):
    n = pl.program_id(0)
    ch = x_vmem.shape[0] // NCHUNK

    def _dot(xs):
        return jax.lax.dot_general(
            xs,
            w_ref[...],
            (((1,), (1,)), ((), ())),
            preferred_element_type=jnp.float32,
        ) * scale_ref[0]

    def _copy(c):
        return pltpu.make_async_copy(
            x_hbm.at[pl.ds(c * ch, ch), :],
            x_vmem.at[pl.ds(c * ch, ch), :],
            sems.at[c],
        )

    @pl.when(n == 0)
    def _():
        for c in range(NCHUNK):
            _copy(c).start()
        for c in range(NCHUNK):
            _copy(c).wait()
            o_ref[pl.ds(c * ch, ch), :] = _dot(x_vmem[pl.ds(c * ch, ch), :])

    @pl.when(n > 0)
    def _():
        o_ref[...] = _dot(x_vmem[...])


@jax.jit
def kernel(x, W, scale):
    M, K = x.shape
    N = W.shape[0]
    return pl.pallas_call(
        _mm_kernel,
        grid_spec=pltpu.PrefetchScalarGridSpec(
            num_scalar_prefetch=1,
            grid=(N // BN,),
            in_specs=[
                pl.BlockSpec(memory_space=pl.ANY),
                pl.BlockSpec((BN, K), lambda n, *_: (n, 0)),
            ],
            out_specs=pl.BlockSpec((M, BN), lambda n, *_: (0, n)),
            scratch_shapes=[
                pltpu.VMEM((M, K), jnp.float32),
                pltpu.SemaphoreType.DMA((NCHUNK,)),
            ],
        ),
        out_shape=jax.ShapeDtypeStruct((M, N), jnp.float32),
        compiler_params=pltpu.CompilerParams(
            dimension_semantics=("arbitrary",),
            vmem_limit_bytes=100 * 1024 * 1024,
        ),
    )(scale, x, W)


# speedup vs baseline: 1.0166x; 1.0001x over previous
import jax
import jax.numpy as jnp
from jax.experimental import pallas as pl
from jax.experimental.pallas import tpu as pltpu

BN = 512
NCHUNK = 4


def _mm_kernel(scale_ref, x_hbm, w_ref, o_ref, x_vmem, sems):
    n = pl.program_id(0)
    ch = x_vmem.shape[0] // NCHUNK

    def _dot(xs):
        return jax.lax.dot_general(
            xs,
            w_ref[...],
            (((1,), (1,)), ((), ())),
            preferred_element_type=jnp.float32,
        )

    def _copy(c):
        return pltpu.make_async_copy(
            x_hbm.at[pl.ds(c * ch, ch), :],
            x_vmem.at[pl.ds(c * ch, ch), :],
            sems.at[c],
        )

    @pl.when(n == 0)
    def _():
        for c in range(NCHUNK):
            _copy(c).start()
        for c in range(NCHUNK):
            _copy(c).wait()
            rows = pl.ds(c * ch, ch)
            x_vmem[rows, :] = x_vmem[rows, :] * scale_ref[0]
            o_ref[rows, :] = _dot(x_vmem[rows, :])

    @pl.when(n > 0)
    def _():
        o_ref[...] = _dot(x_vmem[...])


@jax.jit
def kernel(x, W, scale):
    M, K = x.shape
    N = W.shape[0]
    return pl.pallas_call(
        _mm_kernel,
        grid_spec=pltpu.PrefetchScalarGridSpec(
            num_scalar_prefetch=1,
            grid=(N // BN,),
            in_specs=[
                pl.BlockSpec(memory_space=pl.ANY),
                pl.BlockSpec((BN, K), lambda n, *_: (n, 0)),
            ],
            out_specs=pl.BlockSpec((M, BN), lambda n, *_: (0, n)),
            scratch_shapes=[
                pltpu.VMEM((M, K), jnp.float32),
                pltpu.SemaphoreType.DMA((NCHUNK,)),
            ],
        ),
        out_shape=jax.ShapeDtypeStruct((M, N), jnp.float32),
        compiler_params=pltpu.CompilerParams(
            dimension_semantics=("arbitrary",),
            vmem_limit_bytes=100 * 1024 * 1024,
        ),
    )(scale, x, W)


# speedup vs baseline: 1.0184x; 1.0018x over previous
import jax
import jax.numpy as jnp
from jax.experimental import pallas as pl
from jax.experimental.pallas import tpu as pltpu

BN = 512
NCHUNK = 4


def _mm_kernel(scale_ref, x_hbm, w_ref, o_ref, x_vmem, sems):
    n = pl.program_id(0)
    ch = x_vmem.shape[0] // NCHUNK

    def _dot(xs):
        return jax.lax.dot_general(
            xs,
            w_ref[...],
            (((1,), (1,)), ((), ())),
            preferred_element_type=jnp.float32,
        ) * scale_ref[0]

    def _copy(c):
        return pltpu.make_async_copy(
            x_hbm.at[pl.ds(c * ch, ch), :],
            x_vmem.at[pl.ds(c * ch, ch), :],
            sems.at[c],
        )

    @pl.when(n == 0)
    def _():
        for c in range(NCHUNK):
            _copy(c).start()
        for c in range(NCHUNK):
            _copy(c).wait()
            o_ref[pl.ds(c * ch, ch), :] = _dot(x_vmem[pl.ds(c * ch, ch), :])

    @pl.when(n > 0)
    def _():
        o_ref[...] = _dot(x_vmem[...])


@jax.jit
def kernel(x, W, scale):
    M, K = x.shape
    N = W.shape[0]
    return pl.pallas_call(
        _mm_kernel,
        grid_spec=pltpu.PrefetchScalarGridSpec(
            num_scalar_prefetch=1,
            grid=(N // BN,),
            in_specs=[
                pl.BlockSpec(memory_space=pl.ANY),
                pl.BlockSpec((BN, K), lambda n, *_: (n, 0)),
            ],
            out_specs=pl.BlockSpec((M, BN), lambda n, *_: (0, n)),
            scratch_shapes=[
                pltpu.VMEM((M, K), jnp.float32),
                pltpu.SemaphoreType.DMA((NCHUNK,)),
            ],
        ),
        out_shape=jax.ShapeDtypeStruct((M, N), jnp.float32),
        compiler_params=pltpu.CompilerParams(
            dimension_semantics=("arbitrary",),
            vmem_limit_bytes=100 * 1024 * 1024,
        ),
    )(scale, x, W)
